# initial kernel scaffold (unmeasured)
import jax
import jax.numpy as jnp
from jax import lax
from jax.experimental import pallas as pl
from jax.experimental.pallas import tpu as pltpu

B = 8
H = 8
D = 64
BS = 16
NP_LOCAL = 64
NSLOTS = 64
T = NP_LOCAL * BS


def kernel(Q, K, V, bt, lens):
    lens2 = lens.reshape(B, 1)

    def body(q_ref, k_ref, v_ref, bt_ref, lens_ref, out_ref,
             send_buf, recv_buf, send_sem, recv_sem):
        my_x = lax.axis_index("x")
        my_y = lax.axis_index("y")
        my_z = lax.axis_index("z")
        nbr = (my_x, 1 - my_y, my_z)

        barrier_sem = pltpu.get_barrier_semaphore()
        pl.semaphore_signal(
            barrier_sem, inc=1, device_id=nbr,
            device_id_type=pl.DeviceIdType.MESH,
        )
        pl.semaphore_wait(barrier_sem, 1)

        bt_b = jnp.broadcast_to(bt_ref[:][:, None, :], (B, T, NSLOTS))
        lens_b = jnp.broadcast_to(lens_ref[:][:, None, :], (B, T, NSLOTS))
        page_iota = (
            lax.broadcasted_iota(jnp.int32, (B, T, NSLOTS), 1) // BS
            + my_y * NP_LOCAL
        )
        slot_iota = lax.broadcasted_iota(jnp.int32, (B, T, NSLOTS), 2)
        hit = jnp.logical_and(bt_b == page_iota, slot_iota < lens_b)
        w = jnp.sum(jnp.where(hit, 1.0, 0.0), axis=2)

        q = q_ref[:][:, 0, :, :]
        k = k_ref[:].reshape(T, H, D)
        s = lax.dot_general(
            q, k, (((2,), (2,)), ((1,), (1,))),
            preferred_element_type=jnp.float32,
        ) * (D ** -0.5)

        wv = w[None, :, :]
        s_m = jnp.where(wv > 0, s, -1e30)
        m = jnp.max(s_m, axis=2, keepdims=True)
        p = wv * jnp.exp(s_m - m)
        l = jnp.sum(p, axis=2, keepdims=True)
        v = v_ref[:].reshape(T, H, D)
        acc = lax.dot_general(
            p, v, (((2,), (0,)), ((0,), (1,))),
            preferred_element_type=jnp.float32,
        )

        send_buf[0] = acc
        send_buf[1] = jnp.broadcast_to(m, (H, B, D))
        send_buf[2] = jnp.broadcast_to(l, (H, B, D))
        rdma = pltpu.make_async_remote_copy(
            src_ref=send_buf,
            dst_ref=recv_buf,
            send_sem=send_sem,
            recv_sem=recv_sem,
            device_id=nbr,
            device_id_type=pl.DeviceIdType.MESH,
        )
        rdma.start()
        rdma.wait()

        acc_n = recv_buf[0]
        m_n = recv_buf[1][:, :, 0:1]
        l_n = recv_buf[2][:, :, 0:1]

        m_star = jnp.maximum(m, m_n)
        a_l = jnp.exp(m - m_star)
        a_n = jnp.exp(m_n - m_star)
        out = (acc * a_l + acc_n * a_n) / (l * a_l + l_n * a_n)
        out_ref[...] = jnp.transpose(out, (1, 0, 2))[:, None, :, :]

    return pl.pallas_call(
        body,
        out_shape=jax.ShapeDtypeStruct((B, 1, H, D), jnp.float32),
        in_specs=[pl.BlockSpec(memory_space=pltpu.VMEM)] * 5,
        out_specs=pl.BlockSpec(memory_space=pltpu.VMEM),
        scratch_shapes=[
            pltpu.VMEM((3, H, B, D), jnp.float32),
            pltpu.VMEM((3, H, B, D), jnp.float32),
            pltpu.SemaphoreType.DMA,
            pltpu.SemaphoreType.DMA,
        ],
        compiler_params=pltpu.CompilerParams(collective_id=0),
    )(Q, K, V, bt, lens2)


# baseline (device time: 9315 ns/iter reference)
import jax
import jax.numpy as jnp
from jax import lax
from jax.experimental import pallas as pl
from jax.experimental.pallas import tpu as pltpu

B = 8
H = 8
D = 64
BS = 16
NP_LOCAL = 64
NSLOTS = 64
T = NP_LOCAL * BS


def kernel(Q, K, V, bt, lens):
    lens2 = lens.reshape(B, 1)

    def body(q_ref, k_ref, v_ref, bt_ref, lens_ref, out_ref,
             send_buf, recv_buf, send_sem, recv_sem):
        my_x = lax.axis_index("x")
        my_y = lax.axis_index("y")
        my_z = lax.axis_index("z")
        nbr = (my_x, 1 - my_y, my_z)

        barrier_sem = pltpu.get_barrier_semaphore()
        pl.semaphore_signal(
            barrier_sem, inc=1, device_id=nbr,
            device_id_type=pl.DeviceIdType.MESH,
        )
        pl.semaphore_wait(barrier_sem, 1)

        bt_b = jnp.broadcast_to(bt_ref[:][:, None, :], (B, NP_LOCAL, NSLOTS))
        lens_b = jnp.broadcast_to(lens_ref[:][:, None, :], (B, NP_LOCAL, NSLOTS))
        page_iota = (
            lax.broadcasted_iota(jnp.int32, (B, NP_LOCAL, NSLOTS), 1)
            + my_y * NP_LOCAL
        )
        slot_iota = lax.broadcasted_iota(jnp.int32, (B, NP_LOCAL, NSLOTS), 2)
        hit = jnp.logical_and(bt_b == page_iota, slot_iota < lens_b)
        count = jnp.sum(jnp.where(hit, 1.0, 0.0), axis=2)

        expand = jnp.where(
            lax.broadcasted_iota(jnp.int32, (NP_LOCAL, T), 1) // BS
            == lax.broadcasted_iota(jnp.int32, (NP_LOCAL, T), 0),
            1.0, 0.0,
        )
        w = lax.dot_general(
            count, expand, (((1,), (0,)), ((), ())),
            preferred_element_type=jnp.float32,
        )

        q = q_ref[:][:, 0, :, :]
        k = k_ref[:].reshape(T, H, D)
        s = lax.dot_general(
            q, k, (((2,), (2,)), ((1,), (1,))),
            preferred_element_type=jnp.float32,
        ) * (D ** -0.5)

        wv = w[None, :, :]
        s_m = jnp.where(wv > 0, s, -1e30)
        m = jnp.max(s_m, axis=2, keepdims=True)
        p = wv * jnp.exp(s_m - m)
        l = jnp.sum(p, axis=2, keepdims=True)
        v = v_ref[:].reshape(T, H, D)
        acc = lax.dot_general(
            p, v, (((2,), (0,)), ((0,), (1,))),
            preferred_element_type=jnp.float32,
        )

        send_buf[0] = acc
        send_buf[1] = jnp.broadcast_to(m, (H, B, D))
        send_buf[2] = jnp.broadcast_to(l, (H, B, D))
        rdma = pltpu.make_async_remote_copy(
            src_ref=send_buf,
            dst_ref=recv_buf,
            send_sem=send_sem,
            recv_sem=recv_sem,
            device_id=nbr,
            device_id_type=pl.DeviceIdType.MESH,
        )
        rdma.start()
        rdma.wait()

        acc_n = recv_buf[0]
        m_n = recv_buf[1][:, :, 0:1]
        l_n = recv_buf[2][:, :, 0:1]

        m_star = jnp.maximum(m, m_n)
        a_l = jnp.exp(m - m_star)
        a_n = jnp.exp(m_n - m_star)
        out = (acc * a_l + acc_n * a_n) / (l * a_l + l_n * a_n)
        out_ref[...] = jnp.transpose(out, (1, 0, 2))[:, None, :, :]

    return pl.pallas_call(
        body,
        out_shape=jax.ShapeDtypeStruct((B, 1, H, D), jnp.float32),
        in_specs=[pl.BlockSpec(memory_space=pltpu.VMEM)] * 5,
        out_specs=pl.BlockSpec(memory_space=pltpu.VMEM),
        scratch_shapes=[
            pltpu.VMEM((3, H, B, D), jnp.float32),
            pltpu.VMEM((3, H, B, D), jnp.float32),
            pltpu.SemaphoreType.DMA,
            pltpu.SemaphoreType.DMA,
        ],
        compiler_params=pltpu.CompilerParams(
            collective_id=0, vmem_limit_bytes=100 * 1024 * 1024,
        ),
    )(Q, K, V, bt, lens2)
